# baseline (speedup 1.0000x reference)
import jax
import jax.numpy as jnp
from jax import lax
from jax.experimental import pallas as pl
from jax.experimental.pallas import tpu as pltpu
from jax.experimental.pallas import tpu_sc as plsc

B = 16384
DIM = 64
NC = 2
NS = 16
NW = NC * NS
BPW = B // NW
CH = 64
NCH = BPW // CH
LANES = 16

_LOG1P_COEF = (
    5.621959008883515e-07, 0.999957487075066, -0.49920656854784484,
    0.3269731000138668, -0.22283625832801954, 0.1307650325042385,
    -0.052624851367851076, 0.010119082927824848,
)


def _log1p_poly(t):
    acc = jnp.full_like(t, _LOG1P_COEF[-1])
    for c in reversed(_LOG1P_COEF[:-1]):
        acc = acc * t + jnp.float32(c)
    return acc


def _sc_body(users_hbm, pos_hbm, neg_hbm, utab_hbm, etab_hbm, out_hbm,
             uiv, piv, niv, ubuf, pbuf, nbuf, part,
             usem, psem, nsem):
    wid = lax.axis_index("s") * NC + lax.axis_index("c")
    base = wid * BPW

    pltpu.sync_copy(users_hbm.at[pl.ds(base, BPW)], uiv)
    pltpu.sync_copy(pos_hbm.at[pl.ds(base, BPW)], piv)
    pltpu.sync_copy(neg_hbm.at[pl.ds(base, BPW)], niv)

    lane = lax.iota(jnp.int32, LANES)

    def fire(c, slot):
        def enq(i, _):
            g = c * CH + (i & ~(LANES - 1))
            k = i & (LANES - 1)
            sel = lane == k
            ru = jnp.sum(jnp.where(sel, uiv[pl.ds(g, LANES)], 0), axis=0)
            rp = jnp.sum(jnp.where(sel, piv[pl.ds(g, LANES)], 0), axis=0)
            rn = jnp.sum(jnp.where(sel, niv[pl.ds(g, LANES)], 0), axis=0)
            pltpu.async_copy(utab_hbm.at[pl.ds(ru, 1), :],
                             ubuf.at[slot, pl.ds(i, 1), :], usem)
            pltpu.async_copy(etab_hbm.at[pl.ds(rp, 1), :],
                             pbuf.at[slot, pl.ds(i, 1), :], psem)
            pltpu.async_copy(etab_hbm.at[pl.ds(rn, 1), :],
                             nbuf.at[slot, pl.ds(i, 1), :], nsem)
            return 0
        lax.fori_loop(0, CH, enq, 0)

    def drain(slot):
        pltpu.make_async_copy(utab_hbm.at[pl.ds(0, CH), :],
                              ubuf.at[slot], usem).wait()
        pltpu.make_async_copy(etab_hbm.at[pl.ds(0, CH), :],
                              pbuf.at[slot], psem).wait()
        pltpu.make_async_copy(etab_hbm.at[pl.ds(0, CH), :],
                              nbuf.at[slot], nsem).wait()

    def chunk_sum(slot, acc):
        def row_body(i, a):
            for j in range(DIM // LANES):
                sl = pl.ds(j * LANES, LANES)
                u = ubuf[slot, i, sl]
                p = pbuf[slot, i, sl]
                n = nbuf[slot, i, sl]
                ps = u * p
                ns = u * n
                ea = jnp.exp(-jnp.abs(ps))
                eb = jnp.exp(-jnp.abs(ns))
                a = a + (jnp.maximum(-ps, 0.0) + jnp.maximum(ns, 0.0)
                         + _log1p_poly(ea) + _log1p_poly(eb))
            return a
        return lax.fori_loop(0, CH, row_body, acc)

    acc = jnp.zeros((LANES,), jnp.float32)
    fire(0, 0)
    for c in range(NCH):
        drain(c % 2)
        if c + 1 < NCH:
            fire(c + 1, (c + 1) % 2)
        acc = chunk_sum(c % 2, acc)

    part[...] = acc
    pltpu.sync_copy(part, out_hbm.at[wid])


@jax.jit
def _sc_partials(users, pos, neg, utab, etab):
    mesh = plsc.VectorSubcoreMesh(core_axis_name="c", subcore_axis_name="s")
    f = pl.kernel(
        _sc_body,
        out_type=jax.ShapeDtypeStruct((NW, LANES), jnp.float32),
        mesh=mesh,
        compiler_params=pltpu.CompilerParams(needs_layout_passes=False),
        scratch_types=[
            pltpu.VMEM((BPW,), jnp.int32),
            pltpu.VMEM((BPW,), jnp.int32),
            pltpu.VMEM((BPW,), jnp.int32),
            pltpu.VMEM((2, CH, DIM), jnp.float32),
            pltpu.VMEM((2, CH, DIM), jnp.float32),
            pltpu.VMEM((2, CH, DIM), jnp.float32),
            pltpu.VMEM((LANES,), jnp.float32),
            pltpu.SemaphoreType.DMA,
            pltpu.SemaphoreType.DMA,
            pltpu.SemaphoreType.DMA,
        ],
    )
    return f(users, pos, neg, utab, etab)


def kernel(users, pos, neg, user_table, entity_table):
    parts = _sc_partials(users.astype(jnp.int32), pos.astype(jnp.int32),
                         neg.astype(jnp.int32), user_table, entity_table)
    return jnp.sum(parts) / jnp.float32(B * DIM)
